# Initial kernel scaffold; baseline (speedup 1.0000x reference)
import jax
import jax.numpy as jnp
from jax.experimental import pallas as pl


def kernel(x, edge_attr, edge_index, u, batch, W_e, b_e, W_n, b_n, W_g, b_g):
    raise NotImplementedError("write your pallas kernel here")


# speedup vs baseline: 5.1083x; 5.1083x over previous
import functools

import jax
import jax.numpy as jnp
from jax import lax
from jax.experimental import pallas as pl
from jax.experimental.pallas import tpu as pltpu
from jax.experimental.pallas import tpu_sc as plsc

N_NODES = 10000
N_EDGES = 320000
D_NODE = 128
D_EDGE = 16
D_GLOB = 32
N_GRAPHS = 4

N_WORKERS = 32
EDGES_PER_TILE = N_EDGES // N_WORKERS
CHUNK = 80
N_CHUNKS = EDGES_PER_TILE // CHUNK
ACC_ROWS = 10240
TILE_ROWS = ACC_ROWS // 16

NODE_BLK = 1000
EDGE_BLK = 4000

_F32 = jnp.float32


def _sc_edge_body(a2_hbm, b_hbm, src_hbm, dst_hbm,
                  e_hbm, msg_hbm, cnt_hbm,
                  src_v, dst_v, e_buf, b_v, ones_v, msg_sh, cnt_sh, sem):
    cid = lax.axis_index("c")
    sid = lax.axis_index("s")
    wid = cid * 16 + sid

    zero16 = jnp.zeros((16,), _F32)

    def _z_ones(i, c):
        ones_v[i, :] = zero16
        return c
    lax.fori_loop(0, CHUNK, _z_ones, 0)

    def _z_ebuf(i, c):
        for j in range(8):
            e_buf[i, pl.ds(j * 16, 16)] = zero16
        return c
    lax.fori_loop(0, CHUNK, _z_ebuf, 0)

    rbase = sid * TILE_ROWS
    obase = cid * ACC_ROWS + rbase
    for k in range(8):
        pltpu.sync_copy(e_buf.at[pl.ds(0, 80)],
                        msg_sh.at[pl.ds(rbase + k * 80, 80)])
        pltpu.sync_copy(ones_v.at[pl.ds(0, 80)],
                        cnt_sh.at[pl.ds(rbase + k * 80, 80)])

    lane1 = jnp.where(lax.iota(jnp.int32, 16) == 0,
                      jnp.float32(1.0), jnp.float32(0.0))

    def _s_ones(i, c):
        ones_v[i, :] = lane1
        return c
    lax.fori_loop(0, CHUNK, _s_ones, 0)

    plsc.subcore_barrier()

    ebase = wid * EDGES_PER_TILE

    def _chunk(t, c):
        base = ebase + t * CHUNK
        pltpu.sync_copy(src_hbm.at[pl.ds(base, CHUNK)], src_v)
        pltpu.sync_copy(dst_hbm.at[pl.ds(base, CHUNK)], dst_v.at[0])
        gcp = pltpu.async_copy(a2_hbm.at[src_v], e_buf, sem)
        pltpu.sync_copy(b_hbm.at[pl.ds(base, CHUNK)], b_v)
        gcp.wait()

        def _rows(i, cc):
            for j in range(8):
                v = e_buf[i, pl.ds(j * 16, 16)] + b_v[i, pl.ds(j * 16, 16)]
                e_buf[i, pl.ds(j * 16, 16)] = jnp.maximum(v, 0.0)
            return cc
        lax.fori_loop(0, CHUNK, _rows, 0)

        pltpu.sync_copy(e_buf, e_hbm.at[pl.ds(base, CHUNK)])
        pltpu.sync_copy(e_buf, msg_sh.at[dst_v.at[0]], add=True)
        pltpu.sync_copy(ones_v, cnt_sh.at[dst_v.at[0]], add=True)
        return c

    lax.fori_loop(0, N_CHUNKS, _chunk, 0)

    plsc.subcore_barrier()

    for k in range(8):
        pltpu.sync_copy(msg_sh.at[pl.ds(rbase + k * 80, 80)], e_buf)
        pltpu.sync_copy(e_buf, msg_hbm.at[pl.ds(obase + k * 80, 80)])
        pltpu.sync_copy(cnt_sh.at[pl.ds(rbase + k * 80, 80)], ones_v)
        pltpu.sync_copy(ones_v, cnt_hbm.at[pl.ds(obase + k * 80, 80)])


def _sc_edge(a2, b, src, dst):
    f = pl.kernel(
        _sc_edge_body,
        out_type=[
            jax.ShapeDtypeStruct((N_EDGES, D_NODE), _F32),
            jax.ShapeDtypeStruct((2 * ACC_ROWS, D_NODE), _F32),
            jax.ShapeDtypeStruct((2 * ACC_ROWS, 16), _F32),
        ],
        mesh=plsc.VectorSubcoreMesh(core_axis_name="c", subcore_axis_name="s"),
        compiler_params=pltpu.CompilerParams(use_tc_tiling_on_sc=False),
        scratch_types=[
            pltpu.VMEM((CHUNK,), jnp.int32),
            pltpu.VMEM((1, CHUNK), jnp.int32),
            pltpu.VMEM((CHUNK, D_NODE), _F32),
            pltpu.VMEM((CHUNK, D_NODE), _F32),
            pltpu.VMEM((CHUNK, 16), _F32),
            pltpu.VMEM_SHARED((ACC_ROWS, D_NODE), _F32),
            pltpu.VMEM_SHARED((ACC_ROWS, 16), _F32),
            pltpu.SemaphoreType.DMA,
        ],
    )
    return f(a2, b, src, dst)


def _a2_body(x_ref, bcol_ref, u_ref, we1_ref, we3_ref, be_ref, out_ref):
    oh = (lax.broadcasted_iota(jnp.int32, (NODE_BLK, N_GRAPHS), 1)
          == bcol_ref[...]).astype(_F32)
    uwe3 = jnp.dot(u_ref[...], we3_ref[...], preferred_element_type=_F32)
    out_ref[...] = (jnp.dot(x_ref[...], we1_ref[...], preferred_element_type=_F32)
                    + jnp.dot(oh, uwe3, preferred_element_type=_F32)
                    + be_ref[...])


def _b_body(ea_ref, we2_ref, out_ref):
    out_ref[...] = jnp.dot(ea_ref[...], we2_ref[...], preferred_element_type=_F32)


def _node_body(x_ref, m0_ref, m1_ref, c0_ref, c1_ref, bcol_ref,
               u_ref, wn1_ref, wn2_ref, wn3_ref, bn_ref,
               xout_ref, xg_ref, nc_ref):
    i = pl.program_id(0)
    cnt = c0_ref[:, 0:1] + c1_ref[:, 0:1]
    mean = (m0_ref[...] + m1_ref[...]) / jnp.maximum(cnt, 1.0)
    oh = (lax.broadcasted_iota(jnp.int32, (NODE_BLK, N_GRAPHS), 1)
          == bcol_ref[...]).astype(_F32)
    uwn3 = jnp.dot(u_ref[...], wn3_ref[...], preferred_element_type=_F32)
    acc = (jnp.dot(x_ref[...], wn1_ref[...], preferred_element_type=_F32)
           + jnp.dot(mean, wn2_ref[...], preferred_element_type=_F32)
           + jnp.dot(oh, uwn3, preferred_element_type=_F32)
           + bn_ref[...])
    xo = jnp.maximum(acc, 0.0)
    xout_ref[...] = xo

    @pl.when(i == 0)
    def _():
        xg_ref[...] = jnp.zeros_like(xg_ref)
        nc_ref[...] = jnp.zeros_like(nc_ref)

    dn = (((0,), (0,)), ((), ()))
    xg_ref[...] += lax.dot_general(oh, xo, dn, preferred_element_type=_F32)
    nc_ref[...] += lax.dot_general(oh, jnp.ones_like(xo), dn,
                                   preferred_element_type=_F32)


def _glob_body(e_ref, scol_ref, bcol_ref, xg_ref, nc_ref,
               u_ref, wg1_ref, wg2_ref, wg3_ref, bg_ref,
               uout_ref, eg_acc, ec_acc):
    i = pl.program_id(0)

    @pl.when(i == 0)
    def _():
        eg_acc[...] = jnp.zeros_like(eg_acc)
        ec_acc[...] = jnp.zeros_like(ec_acc)

    bf = bcol_ref[...].astype(_F32)
    c1 = jnp.sum((bf < 1.0).astype(_F32))
    c2 = jnp.sum((bf < 2.0).astype(_F32))
    c3 = jnp.sum((bf < 3.0).astype(_F32))
    sf = scol_ref[...].astype(_F32)
    g = ((sf >= c1).astype(_F32) + (sf >= c2).astype(_F32)
         + (sf >= c3).astype(_F32))
    ohg = (lax.broadcasted_iota(jnp.int32, (EDGE_BLK, N_GRAPHS), 1).astype(_F32)
           == g).astype(_F32)
    dn = (((0,), (0,)), ((), ()))
    eg_acc[...] += lax.dot_general(ohg, e_ref[...], dn, preferred_element_type=_F32)
    ec_acc[...] += lax.dot_general(ohg, jnp.ones_like(e_ref[...]), dn,
                                   preferred_element_type=_F32)

    @pl.when(i == pl.num_programs(0) - 1)
    def _():
        egm = eg_acc[...] / jnp.maximum(ec_acc[...], 1.0)
        xgm = xg_ref[...] / jnp.maximum(nc_ref[...], 1.0)
        uo = (jnp.dot(egm, wg1_ref[...], preferred_element_type=_F32)
              + jnp.dot(xgm, wg2_ref[...], preferred_element_type=_F32)
              + jnp.dot(u_ref[...], wg3_ref[...], preferred_element_type=_F32)
              + bg_ref[...])
        uout_ref[...] = jnp.maximum(uo, 0.0)


def kernel(x, edge_attr, edge_index, u, batch, W_e, b_e, W_n, b_n, W_g, b_g):
    src = edge_index[0].astype(jnp.int32)
    dst = edge_index[1].astype(jnp.int32)
    bcol = batch.astype(jnp.int32).reshape(N_NODES, 1)
    scol = src.reshape(N_EDGES, 1)

    we1 = W_e[:D_NODE]
    we2 = W_e[D_NODE:D_NODE + D_EDGE]
    we3 = W_e[D_NODE + D_EDGE:]
    wn1 = W_n[:D_NODE]
    wn2 = W_n[D_NODE:2 * D_NODE]
    wn3 = W_n[2 * D_NODE:]
    wg1 = W_g[:D_NODE]
    wg2 = W_g[D_NODE:2 * D_NODE]
    wg3 = W_g[2 * D_NODE:]
    be = b_e.reshape(1, D_NODE)
    bn = b_n.reshape(1, D_NODE)
    bg = b_g.reshape(1, D_NODE)

    n_grid = N_NODES // NODE_BLK
    e_grid = N_EDGES // EDGE_BLK

    full = lambda s: pl.BlockSpec(s, lambda i: tuple(0 for _ in s))

    a2 = pl.pallas_call(
        _a2_body,
        grid=(n_grid,),
        in_specs=[
            pl.BlockSpec((NODE_BLK, D_NODE), lambda i: (i, 0)),
            pl.BlockSpec((NODE_BLK, 1), lambda i: (i, 0)),
            full((N_GRAPHS, D_GLOB)),
            full((D_NODE, D_NODE)),
            full((D_GLOB, D_NODE)),
            full((1, D_NODE)),
        ],
        out_specs=pl.BlockSpec((NODE_BLK, D_NODE), lambda i: (i, 0)),
        out_shape=jax.ShapeDtypeStruct((N_NODES, D_NODE), _F32),
    )(x, bcol, u, we1, we3, be)

    b_edges = pl.pallas_call(
        _b_body,
        grid=(e_grid,),
        in_specs=[
            pl.BlockSpec((EDGE_BLK, D_EDGE), lambda i: (i, 0)),
            full((D_EDGE, D_NODE)),
        ],
        out_specs=pl.BlockSpec((EDGE_BLK, D_NODE), lambda i: (i, 0)),
        out_shape=jax.ShapeDtypeStruct((N_EDGES, D_NODE), _F32),
    )(edge_attr, we2)

    e, msg_p, cnt_p = _sc_edge(a2, b_edges, src, dst)
    msg_p = msg_p.reshape(2, ACC_ROWS, D_NODE)[:, :N_NODES]
    cnt_p = cnt_p.reshape(2, ACC_ROWS, 16)[:, :N_NODES]

    x_out, xg, nc = pl.pallas_call(
        _node_body,
        grid=(n_grid,),
        in_specs=[
            pl.BlockSpec((NODE_BLK, D_NODE), lambda i: (i, 0)),
            pl.BlockSpec((NODE_BLK, D_NODE), lambda i: (i, 0)),
            pl.BlockSpec((NODE_BLK, D_NODE), lambda i: (i, 0)),
            pl.BlockSpec((NODE_BLK, 16), lambda i: (i, 0)),
            pl.BlockSpec((NODE_BLK, 16), lambda i: (i, 0)),
            pl.BlockSpec((NODE_BLK, 1), lambda i: (i, 0)),
            full((N_GRAPHS, D_GLOB)),
            full((D_NODE, D_NODE)),
            full((D_NODE, D_NODE)),
            full((D_GLOB, D_NODE)),
            full((1, D_NODE)),
        ],
        out_specs=[
            pl.BlockSpec((NODE_BLK, D_NODE), lambda i: (i, 0)),
            pl.BlockSpec((N_GRAPHS, D_NODE), lambda i: (0, 0)),
            pl.BlockSpec((N_GRAPHS, D_NODE), lambda i: (0, 0)),
        ],
        out_shape=[
            jax.ShapeDtypeStruct((N_NODES, D_NODE), _F32),
            jax.ShapeDtypeStruct((N_GRAPHS, D_NODE), _F32),
            jax.ShapeDtypeStruct((N_GRAPHS, D_NODE), _F32),
        ],
    )(x, msg_p[0], msg_p[1], cnt_p[0], cnt_p[1], bcol, u, wn1, wn2, wn3, bn)

    u_out = pl.pallas_call(
        _glob_body,
        grid=(e_grid,),
        in_specs=[
            pl.BlockSpec((EDGE_BLK, D_NODE), lambda i: (i, 0)),
            pl.BlockSpec((EDGE_BLK, 1), lambda i: (i, 0)),
            full((N_NODES, 1)),
            full((N_GRAPHS, D_NODE)),
            full((N_GRAPHS, D_NODE)),
            full((N_GRAPHS, D_GLOB)),
            full((D_NODE, D_NODE)),
            full((D_NODE, D_NODE)),
            full((D_GLOB, D_NODE)),
            full((1, D_NODE)),
        ],
        out_specs=pl.BlockSpec((N_GRAPHS, D_NODE), lambda i: (0, 0)),
        out_shape=jax.ShapeDtypeStruct((N_GRAPHS, D_NODE), _F32),
        scratch_shapes=[
            pltpu.VMEM((N_GRAPHS, D_NODE), _F32),
            pltpu.VMEM((N_GRAPHS, D_NODE), _F32),
        ],
    )(e, scol, bcol, xg, nc, u, wg1, wg2, wg3, bg)

    return (x_out, e, edge_index, u_out, batch)
